# Initial kernel scaffold; baseline (speedup 1.0000x reference)
import jax
import jax.numpy as jnp
from jax.experimental import pallas as pl


def kernel(user_id, question_id, theta_table, a_table, b_table, c_table):
    raise NotImplementedError("write your pallas kernel here")


# speedup vs baseline: 1.6904x; 1.6904x over previous
import jax
import jax.numpy as jnp
from jax import lax
from jax.experimental import pallas as pl
from jax.experimental.pallas import tpu as pltpu
from jax.experimental.pallas import tpu_sc as plsc

NC = 2
NS = 16
L = 16
NW = NC * NS
BATCH = 16384
BPW = BATCH // NW

VALUE_RANGE = 4.0
A_RANGE = 4.0
DCONST = 1.702


def _irt_body(uid_hbm, qid_hbm, th_hbm, a_hbm, b_hbm, c_hbm, out_hbm,
              uid_v, qid_v, th_v, a_v, b_v, c_v, out_v, sem):
    wid = lax.axis_index("s") * NC + lax.axis_index("c")
    base = wid * BPW

    pltpu.sync_copy(uid_hbm.at[pl.ds(base, BPW)], uid_v)
    pltpu.sync_copy(qid_hbm.at[pl.ds(base, BPW)], qid_v)

    g1 = pltpu.async_copy(th_hbm.at[uid_v], th_v, sem)
    g2 = pltpu.async_copy(a_hbm.at[qid_v], a_v, sem)
    g3 = pltpu.async_copy(b_hbm.at[qid_v], b_v, sem)
    g4 = pltpu.async_copy(c_hbm.at[qid_v], c_v, sem)
    g1.wait()
    g2.wait()
    g3.wait()
    g4.wait()

    one = jnp.full((L,), 1.0, jnp.float32)

    @pl.loop(0, BPW, step=L)
    def _(i):
        sl = pl.ds(i, L)
        th = th_v[sl]
        a = a_v[sl]
        b = b_v[sl]
        c = c_v[sl]
        c_s = one / (one + jnp.exp(-c))
        th_s = VALUE_RANGE * (one / (one + jnp.exp(-th)) - 0.5)
        b_s = VALUE_RANGE * (one / (one + jnp.exp(-b)) - 0.5)
        a_s = A_RANGE / (one + jnp.exp(-a))
        z = one / (one + jnp.exp(-DCONST * a_s * (th_s - b_s)))
        out_v[sl] = c_s + (one - c_s) * z

    pltpu.sync_copy(out_v, out_hbm.at[pl.ds(base, BPW)])


def kernel(user_id, question_id, theta_table, a_table, b_table, c_table):
    uid = user_id.astype(jnp.int32)
    qid = question_id.astype(jnp.int32)
    th = theta_table.reshape(-1)
    a = a_table.reshape(-1)
    b = b_table.reshape(-1)
    c = c_table.reshape(-1)

    mesh = plsc.VectorSubcoreMesh(core_axis_name="c", subcore_axis_name="s")
    f = pl.kernel(
        _irt_body,
        out_type=jax.ShapeDtypeStruct((BATCH,), jnp.float32),
        mesh=mesh,
        scratch_types=[
            pltpu.VMEM((BPW,), jnp.int32),
            pltpu.VMEM((BPW,), jnp.int32),
            pltpu.VMEM((BPW,), jnp.float32),
            pltpu.VMEM((BPW,), jnp.float32),
            pltpu.VMEM((BPW,), jnp.float32),
            pltpu.VMEM((BPW,), jnp.float32),
            pltpu.VMEM((BPW,), jnp.float32),
            pltpu.SemaphoreType.DMA,
        ],
    )
    return f(uid, qid, th, a, b, c)
